# Initial kernel scaffold; baseline (speedup 1.0000x reference)
import jax
import jax.numpy as jnp
from jax.experimental import pallas as pl


def kernel(points, point_indices, memory_bank):
    raise NotImplementedError("write your pallas kernel here")


# speedup vs baseline: 24.1809x; 24.1809x over previous
import jax
import jax.numpy as jnp
import numpy as np
from jax.experimental import pallas as pl
from jax.experimental.pallas import tpu as pltpu

B = 1024
D = 32
M = 100000
K = 4096
BR = 32
NB = B // BR

_I32_MIN = np.int32(np.iinfo(np.int32).min)
_I32_MAX = np.int32(np.iinfo(np.int32).max)
_FLIP = np.int32(0x7FFFFFFF)


def _sortable(bits):
    return jnp.where(bits < 0, bits ^ _FLIP, bits)


def _tc_body(points_ref, idx_ref, mbT_ref, sims_ref, loss_ref, s_ref):
    i = pl.program_id(0)

    p = points_ref[...]
    norm = jnp.sqrt(jnp.sum(p * p, axis=1, keepdims=True))
    pn = p / norm
    sims = jnp.dot(pn, mbT_ref[...], preferred_element_type=jnp.float32)
    sims_ref[...] = sims
    s_ref[...] = _sortable(jax.lax.bitcast_convert_type(sims, jnp.int32))

    lo0 = jnp.full((BR, 1), _I32_MIN, jnp.int32)
    hi0 = jnp.full((BR, 1), _I32_MAX, jnp.int32)

    def body(_, carry):
        lo, hi = carry
        mid = (lo >> 1) + (hi >> 1) + ((lo | hi) & 1)
        cnt = jnp.sum((s_ref[...] >= mid).astype(jnp.int32), axis=1, keepdims=True)
        ge = cnt >= K
        return jnp.where(ge, mid, lo), jnp.where(ge, hi, mid - 1)

    lo, _ = jax.lax.fori_loop(0, 32, body, (lo0, hi0))
    t_bits = lo
    t_f = jax.lax.bitcast_convert_type(_sortable(t_bits), jnp.float32)

    s = s_ref[...]
    gt = s > t_bits
    cnt_gt = jnp.sum(gt.astype(jnp.float32), axis=1, keepdims=True)
    sum_gt = jnp.sum(jnp.where(gt, sims_ref[...], 0.0), axis=1, keepdims=True)
    topk_sum = sum_gt + (np.float32(K) - cnt_gt) * t_f

    idc = idx_ref[0]
    col = jax.lax.broadcasted_iota(jnp.int32, (BR, M), 1)
    pos = jnp.sum(jnp.where(col == idc, sims_ref[...], 0.0), axis=1, keepdims=True)

    part = jnp.sum(-pos + topk_sum * np.float32(1.0 / K), keepdims=True) * np.float32(
        1.0 / B
    )

    @pl.when(i == 0)
    def _():
        loss_ref[...] = jnp.zeros((1, 1), jnp.float32)

    loss_ref[...] += part


def kernel(points, point_indices, memory_bank):
    mbT = memory_bank.T
    idx3 = point_indices.reshape(NB, BR, 1)

    sims, loss = pl.pallas_call(
        _tc_body,
        grid=(NB,),
        in_specs=[
            pl.BlockSpec((BR, D), lambda i: (i, 0)),
            pl.BlockSpec((1, BR, 1), lambda i: (i, 0, 0)),
            pl.BlockSpec((D, M), lambda i: (0, 0)),
        ],
        out_specs=[
            pl.BlockSpec((BR, M), lambda i: (i, 0)),
            pl.BlockSpec((1, 1), lambda i: (0, 0)),
        ],
        out_shape=[
            jax.ShapeDtypeStruct((B, M), jnp.float32),
            jax.ShapeDtypeStruct((1, 1), jnp.float32),
        ],
        scratch_shapes=[pltpu.VMEM((BR, M), jnp.int32)],
    )(points, idx3, mbT)

    return loss[0, 0], sims


# speedup vs baseline: 25.0157x; 1.0345x over previous
import jax
import jax.numpy as jnp
import numpy as np
from jax.experimental import pallas as pl
from jax.experimental.pallas import tpu as pltpu

B = 1024
D = 32
M = 100000
K = 4096
BR = 16
NB = B // BR

W = 4096
NFULL = M // W
TAIL = M - NFULL * W

_FLIP = np.int32(0x7FFFFFFF)


def _sortable(bits):
    return jnp.where(bits < 0, bits ^ _FLIP, bits)


def _count_ge_i16(st_ref, mid):
    mid16 = mid.astype(jnp.int16)

    def chunk(c, acc):
        blk = st_ref[:, pl.ds(c * W, W)]
        return acc + (blk >= mid16).astype(jnp.int16)

    acc = jax.lax.fori_loop(0, NFULL, chunk, jnp.zeros((BR, W), jnp.int16))
    cnt = jnp.sum(acc.astype(jnp.int32), axis=1, keepdims=True)
    tailm = st_ref[:, pl.ds(NFULL * W, TAIL)] >= mid16
    return cnt + jnp.sum(tailm.astype(jnp.int32), axis=1, keepdims=True)


def _search16(st_ref, k):
    lo0 = jnp.full((BR, 1), -32768, jnp.int32)
    hi0 = jnp.full((BR, 1), 32767, jnp.int32)

    def body(_, carry):
        lo, hi = carry
        mid = (lo >> 1) + (hi >> 1) + ((lo | hi) & 1)
        ge = _count_ge_i16(st_ref, mid) >= k
        return jnp.where(ge, mid, lo), jnp.where(ge, hi, mid - 1)

    lo, _ = jax.lax.fori_loop(0, 16, body, (lo0, hi0))
    return lo


def _tc_body(points_ref, idx_ref, mbT_ref, sims_ref, loss_ref, st_ref, mb_vmem, sem):
    i = pl.program_id(0)

    @pl.when(i == 0)
    def _():
        cp = pltpu.make_async_copy(mbT_ref, mb_vmem, sem)
        cp.start()
        cp.wait()

    p = points_ref[...]
    norm = jnp.sqrt(jnp.sum(p * p, axis=1, keepdims=True))
    pn = p / norm
    sims = jnp.dot(pn, mb_vmem[...], preferred_element_type=jnp.float32)
    sims_ref[...] = sims

    s32 = _sortable(jax.lax.bitcast_convert_type(sims, jnp.int32))
    st_ref[...] = (s32 >> 16).astype(jnp.int16)
    kvec = jnp.full((BR, 1), K, jnp.int32)
    p_hi = _search16(st_ref, kvec)

    s32 = _sortable(jax.lax.bitcast_convert_type(sims_ref[...], jnp.int32))
    hi = s32 >> 16
    is_tie = hi == p_hi
    c_hi = jnp.sum((hi > p_hi).astype(jnp.float32), axis=1, keepdims=True).astype(
        jnp.int32
    )
    lo16b = (s32 ^ 0x8000).astype(jnp.int16)
    st_ref[...] = jnp.where(is_tie, lo16b, jnp.int16(-32768))

    t_lo = _search16(st_ref, kvec - c_hi)
    t_bits = (p_hi << 16) | ((t_lo & 0xFFFF) ^ 0x8000)
    t_f = jax.lax.bitcast_convert_type(_sortable(t_bits), jnp.float32)

    simsv = sims_ref[...]
    s32 = _sortable(jax.lax.bitcast_convert_type(simsv, jnp.int32))
    gt = s32 > t_bits
    cnt_gt = jnp.sum(gt.astype(jnp.float32), axis=1, keepdims=True)
    sum_gt = jnp.sum(jnp.where(gt, simsv, 0.0), axis=1, keepdims=True)
    topk_sum = sum_gt + (np.float32(K) - cnt_gt) * t_f

    idc = idx_ref[0]
    col = jax.lax.broadcasted_iota(jnp.int32, (BR, M), 1)
    pos = jnp.sum(jnp.where(col == idc, simsv, 0.0), axis=1, keepdims=True)

    part = jnp.sum(-pos + topk_sum * np.float32(1.0 / K), keepdims=True) * np.float32(
        1.0 / B
    )

    @pl.when(i == 0)
    def _():
        loss_ref[...] = jnp.zeros((1, 1), jnp.float32)

    loss_ref[...] += part


def kernel(points, point_indices, memory_bank):
    mbT = memory_bank.T
    idx3 = point_indices.reshape(NB, BR, 1)

    sims, loss = pl.pallas_call(
        _tc_body,
        grid=(NB,),
        in_specs=[
            pl.BlockSpec((BR, D), lambda i: (i, 0)),
            pl.BlockSpec((1, BR, 1), lambda i: (i, 0, 0)),
            pl.BlockSpec(memory_space=pl.ANY),
        ],
        out_specs=[
            pl.BlockSpec((BR, M), lambda i: (i, 0)),
            pl.BlockSpec((1, 1), lambda i: (0, 0)),
        ],
        out_shape=[
            jax.ShapeDtypeStruct((B, M), jnp.float32),
            jax.ShapeDtypeStruct((1, 1), jnp.float32),
        ],
        scratch_shapes=[
            pltpu.VMEM((BR, M), jnp.int16),
            pltpu.VMEM((D, M), jnp.float32),
            pltpu.SemaphoreType.DMA,
        ],
    )(points, idx3, mbT)

    return loss[0, 0], sims
